# Initial kernel scaffold; baseline (speedup 1.0000x reference)
import jax
import jax.numpy as jnp
from jax.experimental import pallas as pl


def kernel(nodes, to_neighs, feature_table, num_sample):
    raise NotImplementedError("write your pallas kernel here")


# speedup vs baseline: 2.5079x; 2.5079x over previous
import functools

import jax
import jax.numpy as jnp
from jax import lax
from jax.experimental import pallas as pl
from jax.experimental.pallas import tpu as pltpu
from jax.experimental.pallas import tpu_sc as plsc

LANES = 16


def _build_sc_agg(n_nodes, d_feat, b_pad, n_slots, chunk, n_chunks_per_worker,
                  n_workers, inv_count):
    mesh = plsc.VectorSubcoreMesh(core_axis_name="c", subcore_axis_name="s")
    rows_per_worker = n_chunks_per_worker * chunk
    n_vecs = d_feat // LANES

    @functools.partial(
        pl.kernel,
        mesh=mesh,
        out_type=jax.ShapeDtypeStruct((b_pad, d_feat), jnp.float32),
        scratch_types=[
            pltpu.VMEM((n_slots, chunk), jnp.int32),
            pltpu.VMEM((n_slots, chunk, d_feat), jnp.float32),
            pltpu.VMEM((chunk, d_feat), jnp.float32),
            pltpu.SemaphoreType.DMA,
        ],
    )
    def agg(idx_hbm, table_hbm, out_hbm, idx_v, buf_v, outb_v, sem):
        ncores = jax.lax.axis_size("c")
        wid = lax.axis_index("s") * ncores + lax.axis_index("c")
        base_chunk = wid * n_chunks_per_worker

        def chunk_body(c, _):
            chunk_id = base_chunk + c
            row_base = chunk_id * chunk
            pltpu.sync_copy(idx_hbm.at[chunk_id], idx_v)
            copies = []
            for s in range(n_slots):
                copies.append(
                    pltpu.async_copy(table_hbm.at[idx_v.at[s]], buf_v.at[s], sem)
                )
            for cp in copies:
                cp.wait()

            def row_body(r, _):
                for v in range(n_vecs):
                    col = pl.ds(v * LANES, LANES)
                    acc = buf_v[0, r, col]
                    for s in range(1, n_slots):
                        acc = acc + buf_v[s, r, col]
                    outb_v[r, col] = acc * inv_count
                return 0

            lax.fori_loop(0, chunk, row_body, 0)
            pltpu.sync_copy(outb_v, out_hbm.at[pl.ds(row_base, chunk)])
            return 0

        lax.fori_loop(0, n_chunks_per_worker, chunk_body, 0)

    return agg


def kernel(nodes, to_neighs, feature_table, num_sample):
    b = nodes.shape[0]
    sample_width = to_neighs.shape[1]
    n_slots = sample_width + 1
    n_nodes, d_feat = feature_table.shape
    n_workers = 32
    chunk = 64

    n_chunks_per_worker = -(-b // (n_workers * chunk))
    b_pad = n_workers * chunk * n_chunks_per_worker
    inv_count = 1.0 / float(n_slots)

    all_idx = jnp.concatenate([nodes[:, None], to_neighs], axis=1)
    if b_pad != b:
        all_idx = jnp.pad(all_idx, ((0, b_pad - b), (0, 0)))
    idx_blocks = all_idx.reshape(b_pad // chunk, chunk, n_slots).transpose(0, 2, 1)
    idx_blocks = idx_blocks.astype(jnp.int32)

    agg = _build_sc_agg(n_nodes, d_feat, b_pad, n_slots, chunk,
                        n_chunks_per_worker, n_workers, inv_count)
    out = agg(idx_blocks, feature_table)
    return out[:b]


# speedup vs baseline: 2.8538x; 1.1379x over previous
import functools

import jax
import jax.numpy as jnp
from jax import lax
from jax.experimental import pallas as pl
from jax.experimental.pallas import tpu as pltpu
from jax.experimental.pallas import tpu_sc as plsc

LANES = 16


def _build_sc_agg(d_feat, b_pad, n_slots, chunk, n_chunks_pw, n_workers,
                  inv_count):
    mesh = plsc.VectorSubcoreMesh(core_axis_name="c", subcore_axis_name="s")
    rows_per_worker = n_chunks_pw * chunk
    n_vecs = d_feat // LANES
    assert n_chunks_pw % 2 == 0

    @functools.partial(
        pl.kernel,
        mesh=mesh,
        out_type=jax.ShapeDtypeStruct((b_pad, d_feat), jnp.float32),
        scratch_types=[
            pltpu.VMEM((n_chunks_pw * n_slots * chunk,), jnp.int32),
            pltpu.VMEM((2, n_slots, chunk, d_feat), jnp.float32),
            pltpu.VMEM((2, chunk, d_feat), jnp.float32),
            pltpu.SemaphoreType.DMA,
            pltpu.SemaphoreType.DMA,
            pltpu.SemaphoreType.DMA,
            pltpu.SemaphoreType.DMA,
        ],
    )
    def agg(idx_hbm, table_hbm, out_hbm, idx_v, buf_v, outb_v,
            gsem0, gsem1, osem0, osem1):
        ncores = jax.lax.axis_size("c")
        wid = lax.axis_index("s") * ncores + lax.axis_index("c")
        worker_base = wid * rows_per_worker
        gsems = (gsem0, gsem1)
        osems = (osem0, osem1)

        slab = n_slots * chunk * n_chunks_pw
        pltpu.sync_copy(idx_hbm.at[pl.ds(wid * slab, slab)], idx_v)

        def idx_row(k, s):
            return idx_v.at[pl.ds(k * (n_slots * chunk) + s * chunk, chunk)]

        def fire_gathers(k, p, sem):
            for s in range(n_slots):
                pltpu.async_copy(
                    table_hbm.at[idx_row(k, s)], buf_v.at[p].at[s], sem)

        def drain_gathers(k, p, sem):
            for s in range(n_slots):
                pltpu.make_async_copy(
                    table_hbm.at[idx_row(k, s)], buf_v.at[p].at[s], sem
                ).wait()

        def out_slice(k):
            return out_hbm.at[pl.ds(worker_base + k * chunk, chunk)]

        fire_gathers(0, 0, gsems[0])

        def pair_body(i, _):
            for p in (0, 1):
                k = 2 * i + p
                pn = (p + 1) % 2

                @pl.when(k + 1 < n_chunks_pw)
                def _():
                    fire_gathers(k + 1, pn, gsems[pn])

                drain_gathers(k, p, gsems[p])

                @pl.when(k >= 2)
                def _():
                    pltpu.make_async_copy(
                        outb_v.at[p], out_slice(k - 2), osems[p]).wait()

                def row_body(r, _):
                    for v in range(n_vecs):
                        col = pl.ds(v * LANES, LANES)
                        acc = buf_v[p, 0, r, col]
                        for s in range(1, n_slots):
                            acc = acc + buf_v[p, s, r, col]
                        outb_v[p, r, col] = acc * inv_count
                    return 0

                lax.fori_loop(0, chunk, row_body, 0)
                pltpu.async_copy(outb_v.at[p], out_slice(k), osems[p])
            return 0

        lax.fori_loop(0, n_chunks_pw // 2, pair_body, 0)
        for p in (0, 1):
            k = n_chunks_pw - 2 + p
            pltpu.make_async_copy(outb_v.at[p], out_slice(k), osems[p]).wait()

    return agg


def kernel(nodes, to_neighs, feature_table, num_sample):
    b = nodes.shape[0]
    sample_width = to_neighs.shape[1]
    n_slots = sample_width + 1
    n_nodes, d_feat = feature_table.shape
    n_workers = 32
    chunk = 32

    n_chunks_pw = -(-b // (n_workers * chunk))
    n_chunks_pw += n_chunks_pw % 2
    b_pad = n_workers * chunk * n_chunks_pw
    inv_count = 1.0 / float(n_slots)

    all_idx = jnp.concatenate([nodes[:, None], to_neighs], axis=1)
    if b_pad != b:
        all_idx = jnp.pad(all_idx, ((0, b_pad - b), (0, 0)))
    idx_blocks = (all_idx.reshape(n_workers, n_chunks_pw, chunk, n_slots)
                  .transpose(0, 1, 3, 2).reshape(-1).astype(jnp.int32))

    agg = _build_sc_agg(d_feat, b_pad, n_slots, chunk, n_chunks_pw,
                        n_workers, inv_count)
    out = agg(idx_blocks, feature_table)
    return out[:b]
